# Initial kernel scaffold; baseline (speedup 1.0000x reference)
import jax
import jax.numpy as jnp
from jax.experimental import pallas as pl


def kernel(x, edge_index, ptr, W1, b1, W2, b2, Wp, bp):
    raise NotImplementedError("write your pallas kernel here")


# speedup vs baseline: 24.5133x; 24.5133x over previous
import functools

import jax
import jax.numpy as jnp
from jax import lax
from jax.experimental import pallas as pl
from jax.experimental.pallas import tpu as pltpu
from jax.experimental.pallas import tpu_sc as plsc

NC = 2
NS = 16
NW = NC * NS
CH = 128


def _sc_mesh():
    return plsc.VectorSubcoreMesh(core_axis_name="c", subcore_axis_name="s")


def _deg_kernel(nchunk, nacc, dst_hbm, ones_hbm, zeros_hbm, out_hbm,
                dst_v, ones_v, accum):
    cid = lax.axis_index("c")
    sid = lax.axis_index("s")
    wid = sid * NC + cid
    rows = nacc // NS
    pltpu.sync_copy(dst_hbm.at[wid], dst_v)
    pltpu.sync_copy(ones_hbm, ones_v)
    pltpu.sync_copy(zeros_hbm, accum.at[pl.ds(sid * rows, rows)])
    plsc.subcore_barrier()

    def body(j, carry):
        pltpu.sync_copy(ones_v, accum.at[dst_v.at[j]], add=True)
        return carry

    lax.fori_loop(0, nchunk, body, 0)
    plsc.subcore_barrier()
    pltpu.sync_copy(accum.at[pl.ds(sid * rows, rows)],
                    out_hbm.at[cid, pl.ds(sid * rows, rows)])


def _agg_kernel(nchunk, nacc, hp_hbm, src_hbm, dst_hbm, zeros_hbm, out_hbm,
                src_v, dst_v, rows_v, accum, sem):
    cid = lax.axis_index("c")
    sid = lax.axis_index("s")
    wid = sid * NC + cid
    rows = nacc // NS
    pltpu.sync_copy(src_hbm.at[wid], src_v)
    pltpu.sync_copy(dst_hbm.at[wid], dst_v)
    pltpu.sync_copy(zeros_hbm, accum.at[pl.ds(sid * rows, rows)])
    plsc.subcore_barrier()

    def body(j, carry):
        pltpu.async_copy(hp_hbm.at[src_v.at[j]], rows_v, sem).wait()
        pltpu.sync_copy(rows_v, accum.at[dst_v.at[j]], add=True)
        return carry

    lax.fori_loop(0, nchunk, body, 0)
    plsc.subcore_barrier()
    pltpu.sync_copy(accum.at[pl.ds(sid * rows, rows)],
                    out_hbm.at[cid, pl.ds(sid * rows, rows)])


def _stage_a_body(n, x_ref, w1_ref, d0_ref, d1_ref, o_ref):
    deg = d0_ref[:, 0:1] + d1_ref[:, 0:1] + 1.0
    dis = lax.rsqrt(deg[:n])
    hw = jnp.dot(x_ref[...], w1_ref[...], preferred_element_type=jnp.float32)
    o_ref[...] = hw * dis


def _stage_b_body(n, a0_ref, a1_ref, hp_ref, d0_ref, d1_ref, b1_ref, w2_ref,
                  o_ref):
    deg = d0_ref[:, 0:1] + d1_ref[:, 0:1] + 1.0
    dis = lax.rsqrt(deg[:n])
    agg = a0_ref[:n] + a1_ref[:n] + hp_ref[...]
    h1 = jax.nn.relu(agg * dis + b1_ref[...])
    o_ref[...] = jnp.dot(h1, w2_ref[...], preferred_element_type=jnp.float32) * dis


def _stage_c_body(n, g, ptr_ref, a0_ref, a1_ref, hp_ref, d0_ref, d1_ref,
                  b2_ref, wp_ref, bp_ref, o_ref):
    deg = d0_ref[:, 0:1] + d1_ref[:, 0:1] + 1.0
    dis = lax.rsqrt(deg[:n])
    agg = a0_ref[:n] + a1_ref[:n] + hp_ref[...]
    h2 = jax.nn.relu(agg * dis + b2_ref[...])
    idx = lax.broadcasted_iota(jnp.int32, (n, 1), 0)
    means = []
    for gi in range(g):
        lo = ptr_ref[gi]
        hi = ptr_ref[gi + 1]
        m = (idx >= lo) & (idx < hi)
        s = jnp.sum(jnp.where(m, h2, 0.0), axis=0, keepdims=True)
        cnt = jnp.maximum((hi - lo).astype(jnp.float32), 1.0)
        means.append(s / cnt)
    mean = jnp.concatenate(means, axis=0)
    o_ref[...] = jnp.dot(mean, wp_ref[...],
                         preferred_element_type=jnp.float32) + bp_ref[...]


def kernel(x, edge_index, ptr, W1, b1, W2, b2, Wp, bp):
    n, d = x.shape
    h = W1.shape[1]
    g = ptr.shape[0] - 1
    e = edge_index.shape[1]

    ept = -(-e // NW)
    nchunk = -(-ept // CH)
    e_pad = NW * nchunk * CH
    rows_per_tile = ((-(-n // NS)) + 7) // 8 * 8
    nacc = NS * rows_per_tile

    src = jnp.concatenate(
        [edge_index[0], jnp.zeros((e_pad - e,), jnp.int32)]).reshape(
            NW, nchunk, CH)
    dst = jnp.concatenate(
        [edge_index[1], jnp.full((e_pad - e,), n, jnp.int32)]).reshape(
            NW, nchunk, CH)

    ones16 = jnp.ones((CH, 16), jnp.float32)
    zeros16 = jnp.zeros((rows_per_tile, 16), jnp.float32)
    zerosh = jnp.zeros((rows_per_tile, h), jnp.float32)

    deg_call = pl.kernel(
        functools.partial(_deg_kernel, nchunk, nacc),
        out_type=jax.ShapeDtypeStruct((NC, nacc, 16), jnp.float32),
        mesh=_sc_mesh(),
        scratch_types=[
            pltpu.VMEM((nchunk, CH), jnp.int32),
            pltpu.VMEM((CH, 16), jnp.float32),
            pltpu.VMEM_SHARED((nacc, 16), jnp.float32),
        ],
        compiler_params=pltpu.CompilerParams(use_tc_tiling_on_sc=False),
    )
    degp = deg_call(dst, ones16, zeros16)
    d0, d1 = degp[0], degp[1]

    agg_call = pl.kernel(
        functools.partial(_agg_kernel, nchunk, nacc),
        out_type=jax.ShapeDtypeStruct((NC, nacc, h), jnp.float32),
        mesh=_sc_mesh(),
        scratch_types=[
            pltpu.VMEM((nchunk, CH), jnp.int32),
            pltpu.VMEM((nchunk, CH), jnp.int32),
            pltpu.VMEM((CH, h), jnp.float32),
            pltpu.VMEM_SHARED((nacc, h), jnp.float32),
            pltpu.SemaphoreType.DMA,
        ],
        compiler_params=pltpu.CompilerParams(use_tc_tiling_on_sc=False),
    )

    h1p = pl.pallas_call(
        functools.partial(_stage_a_body, n),
        out_shape=jax.ShapeDtypeStruct((n, h), jnp.float32),
    )(x, W1, d0, d1)

    agg1 = agg_call(h1p, src, dst, zerosh)

    h2p = pl.pallas_call(
        functools.partial(_stage_b_body, n),
        out_shape=jax.ShapeDtypeStruct((n, h), jnp.float32),
    )(agg1[0], agg1[1], h1p, d0, d1, b1.reshape(1, h), W2)

    agg2 = agg_call(h2p, src, dst, zerosh)

    out = pl.pallas_call(
        functools.partial(_stage_c_body, n, g),
        out_shape=jax.ShapeDtypeStruct((g, 1), jnp.float32),
        in_specs=[pl.BlockSpec(memory_space=pltpu.SMEM)] +
                 [pl.BlockSpec()] * 8,
    )(ptr, agg2[0], agg2[1], h2p, d0, d1, b2.reshape(1, h), Wp,
      bp.reshape(1, 1))
    return out


# speedup vs baseline: 25.7579x; 1.0508x over previous
import functools

import jax
import jax.numpy as jnp
from jax import lax
from jax.experimental import pallas as pl
from jax.experimental.pallas import tpu as pltpu
from jax.experimental.pallas import tpu_sc as plsc

NC = 2
NS = 16
NW = NC * NS
CH = 128
NBUF = 4


def _sc_mesh():
    return plsc.VectorSubcoreMesh(core_axis_name="c", subcore_axis_name="s")


def _deg_kernel(nchunk, nacc, dst_hbm, ones_hbm, zeros_hbm, out_hbm,
                dst_v, ones_v, accum):
    cid = lax.axis_index("c")
    sid = lax.axis_index("s")
    wid = sid * NC + cid
    rows = nacc // NS
    pltpu.sync_copy(dst_hbm.at[wid], dst_v)
    pltpu.sync_copy(ones_hbm, ones_v)
    pltpu.sync_copy(zeros_hbm, accum.at[pl.ds(sid * rows, rows)])
    plsc.subcore_barrier()

    def body(j, carry):
        pltpu.sync_copy(ones_v, accum.at[dst_v.at[j]], add=True)
        return carry

    lax.fori_loop(0, nchunk, body, 0)
    plsc.subcore_barrier()
    pltpu.sync_copy(accum.at[pl.ds(sid * rows, rows)],
                    out_hbm.at[cid, pl.ds(sid * rows, rows)])


def _agg_kernel(nchunk, nacc, hp_hbm, src_hbm, dst_hbm, zeros_hbm, out_hbm,
                src_v, dst_v, r0, r1, r2, r3, accum,
                g0, g1, g2, g3, s0, s1, s2, s3):
    cid = lax.axis_index("c")
    sid = lax.axis_index("s")
    wid = sid * NC + cid
    rows = nacc // NS
    bufs = (r0, r1, r2, r3)
    gsem = (g0, g1, g2, g3)
    ssem = (s0, s1, s2, s3)
    pltpu.sync_copy(src_hbm.at[wid], src_v)
    pltpu.sync_copy(dst_hbm.at[wid], dst_v)
    pltpu.sync_copy(zeros_hbm, accum.at[pl.ds(sid * rows, rows)])
    plsc.subcore_barrier()

    for b in range(NBUF):
        pltpu.async_copy(hp_hbm.at[src_v.at[b]], bufs[b], gsem[b])

    def body(i, carry):
        base = i * NBUF
        for b in range(NBUF):
            j = base + b
            pltpu.make_async_copy(hp_hbm.at[src_v.at[j]], bufs[b],
                                  gsem[b]).wait()
            pltpu.async_copy(bufs[b], accum.at[dst_v.at[j]], ssem[b],
                             add=True)
        for b in range(NBUF):
            j = base + b
            jn = j + NBUF
            pltpu.make_async_copy(bufs[b], accum.at[dst_v.at[j]],
                                  ssem[b]).wait()

            @pl.when(jn < nchunk)
            def _():
                pltpu.async_copy(hp_hbm.at[src_v.at[jn]], bufs[b], gsem[b])

        return carry

    lax.fori_loop(0, nchunk // NBUF, body, 0)
    plsc.subcore_barrier()
    pltpu.sync_copy(accum.at[pl.ds(sid * rows, rows)],
                    out_hbm.at[cid, pl.ds(sid * rows, rows)])


def _stage_a_body(n, x_ref, w1_ref, d0_ref, d1_ref, o_ref):
    deg = d0_ref[:, 0:1] + d1_ref[:, 0:1] + 1.0
    dis = lax.rsqrt(deg[:n])
    hw = jnp.dot(x_ref[...], w1_ref[...], preferred_element_type=jnp.float32)
    o_ref[...] = hw * dis


def _stage_b_body(n, a0_ref, a1_ref, hp_ref, d0_ref, d1_ref, b1_ref, w2_ref,
                  o_ref):
    deg = d0_ref[:, 0:1] + d1_ref[:, 0:1] + 1.0
    dis = lax.rsqrt(deg[:n])
    agg = a0_ref[:n] + a1_ref[:n] + hp_ref[...]
    h1 = jax.nn.relu(agg * dis + b1_ref[...])
    o_ref[...] = jnp.dot(h1, w2_ref[...], preferred_element_type=jnp.float32) * dis


def _stage_c_body(n, g, ptr_ref, a0_ref, a1_ref, hp_ref, d0_ref, d1_ref,
                  b2_ref, wp_ref, bp_ref, o_ref):
    deg = d0_ref[:, 0:1] + d1_ref[:, 0:1] + 1.0
    dis = lax.rsqrt(deg[:n])
    agg = a0_ref[:n] + a1_ref[:n] + hp_ref[...]
    h2 = jax.nn.relu(agg * dis + b2_ref[...])
    idx = lax.broadcasted_iota(jnp.int32, (n, 1), 0)
    means = []
    for gi in range(g):
        lo = ptr_ref[gi]
        hi = ptr_ref[gi + 1]
        m = (idx >= lo) & (idx < hi)
        s = jnp.sum(jnp.where(m, h2, 0.0), axis=0, keepdims=True)
        cnt = jnp.maximum((hi - lo).astype(jnp.float32), 1.0)
        means.append(s / cnt)
    mean = jnp.concatenate(means, axis=0)
    o_ref[...] = jnp.dot(mean, wp_ref[...],
                         preferred_element_type=jnp.float32) + bp_ref[...]


def kernel(x, edge_index, ptr, W1, b1, W2, b2, Wp, bp):
    n, d = x.shape
    h = W1.shape[1]
    g = ptr.shape[0] - 1
    e = edge_index.shape[1]

    ept = -(-e // NW)
    nchunk = -(-ept // CH)
    nchunk = -(-nchunk // NBUF) * NBUF
    e_pad = NW * nchunk * CH
    rows_per_tile = ((-(-n // NS)) + 7) // 8 * 8
    nacc = NS * rows_per_tile

    src = jnp.concatenate(
        [edge_index[0], jnp.zeros((e_pad - e,), jnp.int32)]).reshape(
            NW, nchunk, CH)
    dst = jnp.concatenate(
        [edge_index[1], jnp.full((e_pad - e,), n, jnp.int32)]).reshape(
            NW, nchunk, CH)

    ones16 = jnp.ones((CH, 16), jnp.float32)
    zeros16 = jnp.zeros((rows_per_tile, 16), jnp.float32)
    zerosh = jnp.zeros((rows_per_tile, h), jnp.float32)

    deg_call = pl.kernel(
        functools.partial(_deg_kernel, nchunk, nacc),
        out_type=jax.ShapeDtypeStruct((NC, nacc, 16), jnp.float32),
        mesh=_sc_mesh(),
        scratch_types=[
            pltpu.VMEM((nchunk, CH), jnp.int32),
            pltpu.VMEM((CH, 16), jnp.float32),
            pltpu.VMEM_SHARED((nacc, 16), jnp.float32),
        ],
        compiler_params=pltpu.CompilerParams(use_tc_tiling_on_sc=False),
    )
    degp = deg_call(dst, ones16, zeros16)
    d0, d1 = degp[0], degp[1]

    agg_call = pl.kernel(
        functools.partial(_agg_kernel, nchunk, nacc),
        out_type=jax.ShapeDtypeStruct((NC, nacc, h), jnp.float32),
        mesh=_sc_mesh(),
        scratch_types=[
            pltpu.VMEM((nchunk, CH), jnp.int32),
            pltpu.VMEM((nchunk, CH), jnp.int32),
        ] + [pltpu.VMEM((CH, h), jnp.float32)] * NBUF + [
            pltpu.VMEM_SHARED((nacc, h), jnp.float32),
        ] + [pltpu.SemaphoreType.DMA] * (2 * NBUF),
        compiler_params=pltpu.CompilerParams(use_tc_tiling_on_sc=False),
    )

    h1p = pl.pallas_call(
        functools.partial(_stage_a_body, n),
        out_shape=jax.ShapeDtypeStruct((n, h), jnp.float32),
    )(x, W1, d0, d1)

    agg1 = agg_call(h1p, src, dst, zerosh)

    h2p = pl.pallas_call(
        functools.partial(_stage_b_body, n),
        out_shape=jax.ShapeDtypeStruct((n, h), jnp.float32),
    )(agg1[0], agg1[1], h1p, d0, d1, b1.reshape(1, h), W2)

    agg2 = agg_call(h2p, src, dst, zerosh)

    out = pl.pallas_call(
        functools.partial(_stage_c_body, n, g),
        out_shape=jax.ShapeDtypeStruct((g, 1), jnp.float32),
        in_specs=[pl.BlockSpec(memory_space=pltpu.SMEM)] +
                 [pl.BlockSpec()] * 8,
    )(ptr, agg2[0], agg2[1], h2p, d0, d1, b2.reshape(1, h), Wp,
      bp.reshape(1, 1))
    return out


# speedup vs baseline: 26.2071x; 1.0174x over previous
import functools

import jax
import jax.numpy as jnp
from jax import lax
from jax.experimental import pallas as pl
from jax.experimental.pallas import tpu as pltpu
from jax.experimental.pallas import tpu_sc as plsc

NC = 2
NS = 16
NW = NC * NS
CH = 128
NBUF = 8


def _sc_mesh():
    return plsc.VectorSubcoreMesh(core_axis_name="c", subcore_axis_name="s")


def _deg_kernel(nchunk, nacc, dst_hbm, ones_hbm, zeros_hbm, out_hbm,
                dst_v, ones_v, accum):
    cid = lax.axis_index("c")
    sid = lax.axis_index("s")
    wid = sid * NC + cid
    rows = nacc // NS
    pltpu.sync_copy(dst_hbm.at[wid], dst_v)
    pltpu.sync_copy(ones_hbm, ones_v)
    pltpu.sync_copy(zeros_hbm, accum.at[pl.ds(sid * rows, rows)])
    plsc.subcore_barrier()

    def body(j, carry):
        pltpu.sync_copy(ones_v, accum.at[dst_v.at[j]], add=True)
        return carry

    lax.fori_loop(0, nchunk, body, 0)
    plsc.subcore_barrier()
    pltpu.sync_copy(accum.at[pl.ds(sid * rows, rows)],
                    out_hbm.at[cid, pl.ds(sid * rows, rows)])


def _agg_kernel(nchunk, nacc, hp_hbm, src_hbm, dst_hbm, zeros_hbm, out_hbm,
                src_v, dst_v, *scratch):
    cid = lax.axis_index("c")
    sid = lax.axis_index("s")
    wid = sid * NC + cid
    rows = nacc // NS
    bufs = scratch[:NBUF]
    accum = scratch[NBUF]
    gsem = scratch[NBUF + 1:2 * NBUF + 1]
    ssem = scratch[2 * NBUF + 1:]
    pltpu.sync_copy(src_hbm.at[wid], src_v)
    pltpu.sync_copy(dst_hbm.at[wid], dst_v)
    pltpu.sync_copy(zeros_hbm, accum.at[pl.ds(sid * rows, rows)])
    plsc.subcore_barrier()

    for b in range(NBUF):
        pltpu.async_copy(hp_hbm.at[src_v.at[b]], bufs[b], gsem[b])

    def body(i, carry):
        base = i * NBUF
        for b in range(NBUF):
            j = base + b
            pltpu.make_async_copy(hp_hbm.at[src_v.at[j]], bufs[b],
                                  gsem[b]).wait()
            pltpu.async_copy(bufs[b], accum.at[dst_v.at[j]], ssem[b],
                             add=True)
        for b in range(NBUF):
            j = base + b
            jn = j + NBUF
            pltpu.make_async_copy(bufs[b], accum.at[dst_v.at[j]],
                                  ssem[b]).wait()

            @pl.when(jn < nchunk)
            def _():
                pltpu.async_copy(hp_hbm.at[src_v.at[jn]], bufs[b], gsem[b])

        return carry

    lax.fori_loop(0, nchunk // NBUF, body, 0)
    plsc.subcore_barrier()
    pltpu.sync_copy(accum.at[pl.ds(sid * rows, rows)],
                    out_hbm.at[cid, pl.ds(sid * rows, rows)])


def _stage_a_body(n, x_ref, w1_ref, d0_ref, d1_ref, o_ref):
    deg = d0_ref[:, 0:1] + d1_ref[:, 0:1] + 1.0
    dis = lax.rsqrt(deg[:n])
    hw = jnp.dot(x_ref[...], w1_ref[...], preferred_element_type=jnp.float32)
    o_ref[...] = hw * dis


def _stage_b_body(n, a0_ref, a1_ref, hp_ref, d0_ref, d1_ref, b1_ref, w2_ref,
                  o_ref):
    deg = d0_ref[:, 0:1] + d1_ref[:, 0:1] + 1.0
    dis = lax.rsqrt(deg[:n])
    agg = a0_ref[:n] + a1_ref[:n] + hp_ref[...]
    h1 = jax.nn.relu(agg * dis + b1_ref[...])
    o_ref[...] = jnp.dot(h1, w2_ref[...], preferred_element_type=jnp.float32) * dis


def _stage_c_body(n, g, ptr_ref, a0_ref, a1_ref, hp_ref, d0_ref, d1_ref,
                  b2_ref, wp_ref, bp_ref, o_ref):
    deg = d0_ref[:, 0:1] + d1_ref[:, 0:1] + 1.0
    dis = lax.rsqrt(deg[:n])
    agg = a0_ref[:n] + a1_ref[:n] + hp_ref[...]
    h2 = jax.nn.relu(agg * dis + b2_ref[...])
    idx = lax.broadcasted_iota(jnp.int32, (n, 1), 0)
    means = []
    for gi in range(g):
        lo = ptr_ref[gi]
        hi = ptr_ref[gi + 1]
        m = (idx >= lo) & (idx < hi)
        s = jnp.sum(jnp.where(m, h2, 0.0), axis=0, keepdims=True)
        cnt = jnp.maximum((hi - lo).astype(jnp.float32), 1.0)
        means.append(s / cnt)
    mean = jnp.concatenate(means, axis=0)
    o_ref[...] = jnp.dot(mean, wp_ref[...],
                         preferred_element_type=jnp.float32) + bp_ref[...]


def kernel(x, edge_index, ptr, W1, b1, W2, b2, Wp, bp):
    n, d = x.shape
    h = W1.shape[1]
    g = ptr.shape[0] - 1
    e = edge_index.shape[1]

    ept = -(-e // NW)
    nchunk = -(-ept // CH)
    nchunk = -(-nchunk // NBUF) * NBUF
    e_pad = NW * nchunk * CH
    rows_per_tile = ((-(-n // NS)) + 7) // 8 * 8
    nacc = NS * rows_per_tile

    src = jnp.concatenate(
        [edge_index[0], jnp.zeros((e_pad - e,), jnp.int32)]).reshape(
            NW, nchunk, CH)
    dst = jnp.concatenate(
        [edge_index[1], jnp.full((e_pad - e,), n, jnp.int32)]).reshape(
            NW, nchunk, CH)

    ones16 = jnp.ones((CH, 16), jnp.float32)
    zeros16 = jnp.zeros((rows_per_tile, 16), jnp.float32)
    zerosh = jnp.zeros((rows_per_tile, h), jnp.float32)

    deg_call = pl.kernel(
        functools.partial(_deg_kernel, nchunk, nacc),
        out_type=jax.ShapeDtypeStruct((NC, nacc, 16), jnp.float32),
        mesh=_sc_mesh(),
        scratch_types=[
            pltpu.VMEM((nchunk, CH), jnp.int32),
            pltpu.VMEM((CH, 16), jnp.float32),
            pltpu.VMEM_SHARED((nacc, 16), jnp.float32),
        ],
        compiler_params=pltpu.CompilerParams(use_tc_tiling_on_sc=False),
    )
    degp = deg_call(dst, ones16, zeros16)
    d0, d1 = degp[0], degp[1]

    agg_call = pl.kernel(
        functools.partial(_agg_kernel, nchunk, nacc),
        out_type=jax.ShapeDtypeStruct((NC, nacc, h), jnp.float32),
        mesh=_sc_mesh(),
        scratch_types=[
            pltpu.VMEM((nchunk, CH), jnp.int32),
            pltpu.VMEM((nchunk, CH), jnp.int32),
        ] + [pltpu.VMEM((CH, h), jnp.float32)] * NBUF + [
            pltpu.VMEM_SHARED((nacc, h), jnp.float32),
        ] + [pltpu.SemaphoreType.DMA] * (2 * NBUF),
        compiler_params=pltpu.CompilerParams(use_tc_tiling_on_sc=False),
    )

    h1p = pl.pallas_call(
        functools.partial(_stage_a_body, n),
        out_shape=jax.ShapeDtypeStruct((n, h), jnp.float32),
    )(x, W1, d0, d1)

    agg1 = agg_call(h1p, src, dst, zerosh)

    h2p = pl.pallas_call(
        functools.partial(_stage_b_body, n),
        out_shape=jax.ShapeDtypeStruct((n, h), jnp.float32),
    )(agg1[0], agg1[1], h1p, d0, d1, b1.reshape(1, h), W2)

    agg2 = agg_call(h2p, src, dst, zerosh)

    out = pl.pallas_call(
        functools.partial(_stage_c_body, n, g),
        out_shape=jax.ShapeDtypeStruct((g, 1), jnp.float32),
        in_specs=[pl.BlockSpec(memory_space=pltpu.SMEM)] +
                 [pl.BlockSpec()] * 8,
    )(ptr, agg2[0], agg2[1], h2p, d0, d1, b2.reshape(1, h), Wp,
      bp.reshape(1, 1))
    return out
